# speedup vs baseline: 92.7263x; 92.7263x over previous
import functools

import jax
import jax.numpy as jnp
from jax import lax
from jax.experimental import pallas as pl
from jax.experimental.pallas import tpu as pltpu
from jax.experimental.pallas import tpu_sc as plsc

_N = 2048
_M = _N * (_N - 1) // 2
_F = 16
_PAD = _M + 48
_CHUNK = _N + 16
_NC = 2
_NS = 16
_NW = _NC * _NS
_RPW = _N // _NW
_B = 256


def _phase1_body(pp_hbm, u_hbm, inbuf, outbuf):
    wid = lax.axis_index("s") * _NC + lax.axis_index("c")

    def row_body(t, carry):
        i = wid * _RPW + t
        off = i * (_N - 1) - (i * (i - 1)) // 2
        start = _F + off - i - 1
        a16 = pl.multiple_of((start // 16) * 16, 16)
        r = start - a16
        pltpu.sync_copy(pp_hbm.at[pl.ds(a16, _CHUNK)], inbuf)

        def shift_body(k, c2):
            outbuf[pl.ds(k * 16, 16)] = inbuf[pl.ds(r + k * 16, 16)]
            return c2

        lax.fori_loop(0, _N // 16, shift_body, 0)
        pltpu.sync_copy(outbuf, u_hbm.at[i])
        return carry

    lax.fori_loop(0, _RPW, row_body, 0)


@functools.partial(
    pl.kernel,
    out_type=jax.ShapeDtypeStruct((_N, _N), jnp.float32),
    mesh=plsc.VectorSubcoreMesh(core_axis_name="c", subcore_axis_name="s"),
    scratch_types=[
        pltpu.VMEM((_CHUNK,), jnp.float32),
        pltpu.VMEM((_N,), jnp.float32),
    ],
)
def _phase1(pp_hbm, u_hbm, inbuf, outbuf):
    _phase1_body(pp_hbm, u_hbm, inbuf, outbuf)


def _phase2_body(ua_ref, ub_ref, o_ref):
    bi = pl.program_id(0)
    bj = pl.program_id(1)
    gr = lax.broadcasted_iota(jnp.int32, (_B, _B), 0) + bi * _B
    gc = lax.broadcasted_iota(jnp.int32, (_B, _B), 1) + bj * _B
    ua = ua_ref[...]
    ubt = ub_ref[...].T
    zero = jnp.float32(0.0)
    o_ref[...] = jnp.where(gc > gr, ua, zero) - jnp.where(gc < gr, ubt, zero)


def _phase2(u):
    return pl.pallas_call(
        _phase2_body,
        grid=(_N // _B, _N // _B),
        in_specs=[
            pl.BlockSpec((_B, _B), lambda i, j: (i, j)),
            pl.BlockSpec((_B, _B), lambda i, j: (j, i)),
        ],
        out_specs=pl.BlockSpec((_B, _B), lambda i, j: (i, j)),
        out_shape=jax.ShapeDtypeStruct((_N, _N), jnp.float32),
    )(u, u)


def kernel(params, triu_indices):
    del triu_indices
    pp = jnp.pad(params.astype(jnp.float32), (_F, _PAD - _M - _F))
    u = _phase1(pp)
    return _phase2(u)


# speedup vs baseline: 148.0734x; 1.5969x over previous
import functools

import jax
import jax.numpy as jnp
import numpy as np
from jax import lax
from jax.experimental import pallas as pl
from jax.experimental.pallas import tpu as pltpu
from jax.experimental.pallas import tpu_sc as plsc

_N = 2048
_M = _N * (_N - 1) // 2
_F = 8
_PAD = _M + 24
_CHUNK = _N + 8
_NC = 2
_NS = 16
_NW = _NC * _NS
_RPW = _N // _NW
_B = 256
_NB = _N // _B
_NPAIR = _NB * (_NB + 1) // 2


def _phase1_body(pp_hbm, u_hbm, in0, in1, ob0, ob1, si0, si1, so0, so1):
    wid = lax.axis_index("s") * _NC + lax.axis_index("c")

    def row_params(t):
        i = t * _NW + wid
        off = i * (_N - 1) - (i * (i - 1)) // 2
        start = _F + off - i - 1
        a8 = pl.multiple_of((start // 8) * 8, 8)
        r = start - a8
        return i, a8, r

    def in_copy(t, buf, sem):
        _, a8, _ = row_params(t)
        return pltpu.make_async_copy(pp_hbm.at[pl.ds(a8, _CHUNK)], buf, sem)

    def out_copy(t, buf, sem):
        i, _, _ = row_params(t)
        return pltpu.make_async_copy(buf, u_hbm.at[i], sem)

    def shift(t, ib, ob):
        i, _, r = row_params(t)
        kb0 = (i + 1) // 128

        def blk(kb, c2):
            base = kb * 128
            for u in range(8):
                o = base + u * 16
                ob[pl.ds(o, 16)] = ib[pl.ds(r + o, 16)]
            return c2

        lax.fori_loop(kb0, _N // 128, blk, 0)

    slots = ((in0, si0, ob0, so0), (in1, si1, ob1, so1))

    in_copy(0, in0, si0).start()

    def body(t2, carry):
        for s in (0, 1):
            t = 2 * t2 + s
            ib, isem, ob, osem = slots[s]
            nib, nisem = slots[1 - s][0], slots[1 - s][1]

            @pl.when(t + 1 < _RPW)
            def _():
                in_copy(t + 1, nib, nisem).start()

            in_copy(t, ib, isem).wait()

            @pl.when(t >= 2)
            def _():
                out_copy(t - 2, ob, osem).wait()

            shift(t, ib, ob)
            out_copy(t, ob, osem).start()
        return carry

    lax.fori_loop(0, _RPW // 2, body, 0)
    out_copy(_RPW - 2, ob0, so0).wait()
    out_copy(_RPW - 1, ob1, so1).wait()


@functools.partial(
    pl.kernel,
    out_type=jax.ShapeDtypeStruct((_N, _N), jnp.float32),
    mesh=plsc.VectorSubcoreMesh(core_axis_name="c", subcore_axis_name="s"),
    scratch_types=[
        pltpu.VMEM((_CHUNK,), jnp.float32),
        pltpu.VMEM((_CHUNK,), jnp.float32),
        pltpu.VMEM((_N,), jnp.float32),
        pltpu.VMEM((_N,), jnp.float32),
        pltpu.SemaphoreType.DMA,
        pltpu.SemaphoreType.DMA,
        pltpu.SemaphoreType.DMA,
        pltpu.SemaphoreType.DMA,
    ],
)
def _phase1(pp_hbm, u_hbm, in0, in1, ob0, ob1, si0, si1, so0, so1):
    _phase1_body(pp_hbm, u_hbm, in0, in1, ob0, ob1, si0, si1, so0, so1)


def _phase2_body(bi_ref, bj_ref, u_ref, o_ref):
    g = pl.program_id(0)
    h = pl.program_id(1)
    bi = bi_ref[g]
    bj = bj_ref[g]
    ua = u_ref[...]
    ir = lax.broadcasted_iota(jnp.int32, (_B, _B), 0)
    ic = lax.broadcasted_iota(jnp.int32, (_B, _B), 1)
    zero = jnp.float32(0.0)

    @pl.when(h == 0)
    def _():
        gr = ir + bi * _B
        gc = ic + bj * _B
        o_ref[...] = jnp.where(gc > gr, ua, zero)

    @pl.when(h == 1)
    def _():
        gr = ir + bj * _B
        gc = ic + bi * _B
        uat = ua.T
        o_ref[...] = jnp.where(gc < gr, -uat, jnp.where(gc > gr, ua, zero))


def _phase2(u):
    pairs = [(a, b) for a in range(_NB) for b in range(a, _NB)]
    bi = jnp.asarray(np.array([p[0] for p in pairs], dtype=np.int32))
    bj = jnp.asarray(np.array([p[1] for p in pairs], dtype=np.int32))
    grid_spec = pltpu.PrefetchScalarGridSpec(
        num_scalar_prefetch=2,
        grid=(_NPAIR, 2),
        in_specs=[
            pl.BlockSpec((_B, _B), lambda g, h, bi, bj: (bi[g], bj[g])),
        ],
        out_specs=pl.BlockSpec(
            (_B, _B),
            lambda g, h, bi, bj: (
                jnp.where(h == 0, bi[g], bj[g]),
                jnp.where(h == 0, bj[g], bi[g]),
            ),
        ),
    )
    return pl.pallas_call(
        _phase2_body,
        grid_spec=grid_spec,
        out_shape=jax.ShapeDtypeStruct((_N, _N), jnp.float32),
    )(bi, bj, u)


def kernel(params, triu_indices):
    del triu_indices
    pp = jnp.pad(params.astype(jnp.float32), (_F, _PAD - _M - _F))
    u = _phase1(pp)
    return _phase2(u)


# speedup vs baseline: 198.0081x; 1.3372x over previous
import functools

import jax
import jax.numpy as jnp
import numpy as np
from jax import lax
from jax.experimental import pallas as pl
from jax.experimental.pallas import tpu as pltpu
from jax.experimental.pallas import tpu_sc as plsc

_N = 2048
_M = _N * (_N - 1) // 2
_F = 8
_PAD = _M + 24
_CHUNK = _N + 8
_NC = 2
_NS = 16
_NW = _NC * _NS
_RPW = _N // _NW
_B = 512
_NB = _N // _B
_NPAIR = _NB * (_NB + 1) // 2


def _phase1_body(pp_hbm, u_hbm, in0, in1, ob0, ob1, si0, si1, so0, so1):
    wid = lax.axis_index("s") * _NC + lax.axis_index("c")

    def row_params(t):
        i = t * _NW + wid
        off = i * (_N - 1) - (i * (i - 1)) // 2
        start = _F + off - i - 1
        a8 = pl.multiple_of((start // 8) * 8, 8)
        r = start - a8
        return i, a8, r

    def in_copy(t, buf, sem):
        _, a8, _ = row_params(t)
        return pltpu.make_async_copy(pp_hbm.at[pl.ds(a8, _CHUNK)], buf, sem)

    def out_copy(t, buf, sem):
        i, _, _ = row_params(t)
        return pltpu.make_async_copy(buf, u_hbm.at[i], sem)

    def shift(t, ib, ob):
        i, _, r = row_params(t)
        kb0 = (i + 1) // 256

        def blk(kb, c2):
            base = kb * 256
            vals = [ib[pl.ds(r + base + u * 16, 16)] for u in range(16)]
            for u, v in enumerate(vals):
                ob[pl.ds(base + u * 16, 16)] = v
            return c2

        lax.fori_loop(kb0, _N // 256, blk, 0)

    slots = ((in0, si0, ob0, so0), (in1, si1, ob1, so1))

    in_copy(0, in0, si0).start()

    def body(t2, carry):
        for s in (0, 1):
            t = 2 * t2 + s
            ib, isem, ob, osem = slots[s]
            nib, nisem = slots[1 - s][0], slots[1 - s][1]

            @pl.when(t + 1 < _RPW)
            def _():
                in_copy(t + 1, nib, nisem).start()

            in_copy(t, ib, isem).wait()

            @pl.when(t >= 2)
            def _():
                out_copy(t - 2, ob, osem).wait()

            shift(t, ib, ob)
            out_copy(t, ob, osem).start()
        return carry

    lax.fori_loop(0, _RPW // 2, body, 0)
    out_copy(_RPW - 2, ob0, so0).wait()
    out_copy(_RPW - 1, ob1, so1).wait()


@functools.partial(
    pl.kernel,
    out_type=jax.ShapeDtypeStruct((_N, _N), jnp.float32),
    mesh=plsc.VectorSubcoreMesh(core_axis_name="c", subcore_axis_name="s"),
    scratch_types=[
        pltpu.VMEM((_CHUNK,), jnp.float32),
        pltpu.VMEM((_CHUNK,), jnp.float32),
        pltpu.VMEM((_N,), jnp.float32),
        pltpu.VMEM((_N,), jnp.float32),
        pltpu.SemaphoreType.DMA,
        pltpu.SemaphoreType.DMA,
        pltpu.SemaphoreType.DMA,
        pltpu.SemaphoreType.DMA,
    ],
)
def _phase1(pp_hbm, u_hbm, in0, in1, ob0, ob1, si0, si1, so0, so1):
    _phase1_body(pp_hbm, u_hbm, in0, in1, ob0, ob1, si0, si1, so0, so1)


def _phase2_body(bi_ref, bj_ref, u_ref, o_ref):
    g = pl.program_id(0)
    h = pl.program_id(1)
    bi = bi_ref[g]
    bj = bj_ref[g]
    ua = u_ref[...]
    ir = lax.broadcasted_iota(jnp.int32, (_B, _B), 0)
    ic = lax.broadcasted_iota(jnp.int32, (_B, _B), 1)
    zero = jnp.float32(0.0)

    @pl.when(h == 0)
    def _():
        gr = ir + bi * _B
        gc = ic + bj * _B
        o_ref[...] = jnp.where(gc > gr, ua, zero)

    @pl.when(h == 1)
    def _():
        gr = ir + bj * _B
        gc = ic + bi * _B
        uat = ua.T
        o_ref[...] = jnp.where(gc < gr, -uat, jnp.where(gc > gr, ua, zero))


def _phase2(u):
    pairs = [(a, b) for a in range(_NB) for b in range(a, _NB)]
    bi = jnp.asarray(np.array([p[0] for p in pairs], dtype=np.int32))
    bj = jnp.asarray(np.array([p[1] for p in pairs], dtype=np.int32))
    grid_spec = pltpu.PrefetchScalarGridSpec(
        num_scalar_prefetch=2,
        grid=(_NPAIR, 2),
        in_specs=[
            pl.BlockSpec((_B, _B), lambda g, h, bi, bj: (bi[g], bj[g])),
        ],
        out_specs=pl.BlockSpec(
            (_B, _B),
            lambda g, h, bi, bj: (
                jnp.where(h == 0, bi[g], bj[g]),
                jnp.where(h == 0, bj[g], bi[g]),
            ),
        ),
    )
    return pl.pallas_call(
        _phase2_body,
        grid_spec=grid_spec,
        out_shape=jax.ShapeDtypeStruct((_N, _N), jnp.float32),
    )(bi, bj, u)


def kernel(params, triu_indices):
    del triu_indices
    pp = jnp.pad(params.astype(jnp.float32), (_F, _PAD - _M - _F))
    u = _phase1(pp)
    return _phase2(u)


# speedup vs baseline: 242.2322x; 1.2233x over previous
import functools

import jax
import jax.numpy as jnp
import numpy as np
from jax import lax
from jax.experimental import pallas as pl
from jax.experimental.pallas import tpu as pltpu
from jax.experimental.pallas import tpu_sc as plsc

_N = 2048
_H = _N // 2
_M = _N * (_N - 1) // 2
_F = 8
_PAD = _M + 24
_CHUNK = _N + 8
_NC = 2
_NS = 16
_NW = _NC * _NS
_RPW = _N // _NW
_DEPTH = 4
_B = 512
_NB = _N // _B
_NPAIR = _NB * (_NB + 1) // 2
_THALF = _H // _NW


def _phase1_body(pp_hbm, u_hbm, ins, obs, sis, sos):
    wid = lax.axis_index("s") * _NC + lax.axis_index("c")

    def row_params(t):
        i = t * _NW + wid
        off = i * (_N - 1) - (i * (i - 1)) // 2
        start = _F + off - i - 1
        a8 = pl.multiple_of((start // 8) * 8, 8)
        r = start - a8
        return i, a8, r

    def in_full(t, s):
        _, a8, _ = row_params(t)
        return pltpu.make_async_copy(
            pp_hbm.at[pl.ds(a8, _CHUNK)], ins[s], sis[s])

    def in_half(t, s):
        _, a8, _ = row_params(t)
        return pltpu.make_async_copy(
            pp_hbm.at[pl.ds(a8 + _H, _H + 8)],
            ins[s].at[pl.ds(_H, _H + 8)], sis[s])

    def out_full(t, s):
        i, _, _ = row_params(t)
        return pltpu.make_async_copy(obs[s], u_hbm.at[i], sos[s])

    def out_half(t, s):
        i, _, _ = row_params(t)
        return pltpu.make_async_copy(
            obs[s].at[pl.ds(_H, _H)], u_hbm.at[i, pl.ds(_H, _H)], sos[s])

    def in_start(t, s):
        @pl.when(t < _THALF)
        def _():
            in_full(t, s).start()

        @pl.when(t >= _THALF)
        def _():
            in_half(t, s).start()

    def in_wait(t, s):
        @pl.when(t < _THALF)
        def _():
            in_full(t, s).wait()

        @pl.when(t >= _THALF)
        def _():
            in_half(t, s).wait()

    def out_start(t, s):
        @pl.when(t < _THALF)
        def _():
            out_full(t, s).start()

        @pl.when(t >= _THALF)
        def _():
            out_half(t, s).start()

    def out_wait(t, s):
        @pl.when(t < _THALF)
        def _():
            out_full(t, s).wait()

        @pl.when(t >= _THALF)
        def _():
            out_half(t, s).wait()

    def shift(t, s):
        i, _, r = row_params(t)
        ib = ins[s]
        ob = obs[s]
        kb0 = (i + 1) // 256

        def blk(kb, c2):
            base = kb * 256
            vals = [ib[pl.ds(r + base + u * 16, 16)] for u in range(16)]
            for u, v in enumerate(vals):
                ob[pl.ds(base + u * 16, 16)] = v
            return c2

        lax.fori_loop(kb0, _N // 256, blk, 0)

    for t in range(_DEPTH - 1):
        in_start(t, t)

    def body(t4, carry):
        for s in range(_DEPTH):
            t = _DEPTH * t4 + s

            @pl.when(t + _DEPTH - 1 < _RPW)
            def _():
                in_start(t + _DEPTH - 1, (s + _DEPTH - 1) % _DEPTH)

            in_wait(t, s)

            @pl.when(t >= _DEPTH)
            def _():
                out_wait(t - _DEPTH, s)

            shift(t, s)
            out_start(t, s)
        return carry

    lax.fori_loop(0, _RPW // _DEPTH, body, 0)
    for t in range(_RPW - _DEPTH, _RPW):
        out_wait(t, t % _DEPTH)


@functools.partial(
    pl.kernel,
    out_type=jax.ShapeDtypeStruct((_N, _N), jnp.float32),
    mesh=plsc.VectorSubcoreMesh(core_axis_name="c", subcore_axis_name="s"),
    scratch_types=(
        [pltpu.VMEM((_CHUNK,), jnp.float32)] * _DEPTH
        + [pltpu.VMEM((_N,), jnp.float32)] * _DEPTH
        + [pltpu.SemaphoreType.DMA] * (2 * _DEPTH)
    ),
)
def _phase1(pp_hbm, u_hbm, *bufs):
    ins = bufs[0:_DEPTH]
    obs = bufs[_DEPTH:2 * _DEPTH]
    sis = bufs[2 * _DEPTH:3 * _DEPTH]
    sos = bufs[3 * _DEPTH:4 * _DEPTH]
    _phase1_body(pp_hbm, u_hbm, ins, obs, sis, sos)


def _phase2_body(bi_ref, bj_ref, u_ref, o_ref):
    g = pl.program_id(0)
    h = pl.program_id(1)
    bi = bi_ref[g]
    bj = bj_ref[g]
    ua = u_ref[...]
    ir = lax.broadcasted_iota(jnp.int32, (_B, _B), 0)
    ic = lax.broadcasted_iota(jnp.int32, (_B, _B), 1)
    zero = jnp.float32(0.0)

    @pl.when(h == 0)
    def _():
        gr = ir + bi * _B
        gc = ic + bj * _B
        o_ref[...] = jnp.where(gc > gr, ua, zero)

    @pl.when(h == 1)
    def _():
        gr = ir + bj * _B
        gc = ic + bi * _B
        uat = ua.T
        o_ref[...] = jnp.where(gc < gr, -uat, jnp.where(gc > gr, ua, zero))


def _phase2(u):
    pairs = [(a, b) for a in range(_NB) for b in range(a, _NB)]
    bi = jnp.asarray(np.array([p[0] for p in pairs], dtype=np.int32))
    bj = jnp.asarray(np.array([p[1] for p in pairs], dtype=np.int32))
    grid_spec = pltpu.PrefetchScalarGridSpec(
        num_scalar_prefetch=2,
        grid=(_NPAIR, 2),
        in_specs=[
            pl.BlockSpec((_B, _B), lambda g, h, bi, bj: (bi[g], bj[g])),
        ],
        out_specs=pl.BlockSpec(
            (_B, _B),
            lambda g, h, bi, bj: (
                jnp.where(h == 0, bi[g], bj[g]),
                jnp.where(h == 0, bj[g], bi[g]),
            ),
        ),
    )
    return pl.pallas_call(
        _phase2_body,
        grid_spec=grid_spec,
        out_shape=jax.ShapeDtypeStruct((_N, _N), jnp.float32),
    )(bi, bj, u)


def kernel(params, triu_indices):
    del triu_indices
    pp = jnp.pad(params.astype(jnp.float32), (_F, _PAD - _M - _F))
    u = _phase1(pp)
    return _phase2(u)


# speedup vs baseline: 257.6729x; 1.0637x over previous
import functools

import jax
import jax.numpy as jnp
import numpy as np
from jax import lax
from jax.experimental import pallas as pl
from jax.experimental.pallas import tpu as pltpu
from jax.experimental.pallas import tpu_sc as plsc

_N = 2048
_H = _N // 2
_M = _N * (_N - 1) // 2
_CHUNK = _N + 8
_NC = 2
_NS = 16
_NW = _NC * _NS
_RPW = _N // _NW
_DEPTH = 4
_B = 512
_NB = _N // _B
_NPAIR = _NB * (_NB + 1) // 2
_THALF = _H // _NW


def _phase1_body(pp_hbm, u_hbm, ins, obs, sis, sos):
    wid = lax.axis_index("s") * _NC + lax.axis_index("c")

    def row_params(t):
        i = t * _NW + wid
        off = i * (_N - 1) - (i * (i - 1)) // 2
        start = off - i - 1
        a8 = pl.multiple_of(
            jnp.clip((start // 8) * 8, 0, _M - _CHUNK).astype(jnp.int32), 8)
        r = jnp.where(i == 0, 7, start - a8)
        return i, a8, r

    def in_full(t, s):
        i, a8, _ = row_params(t)

        @pl.when(i == 0)
        def _():
            pltpu.make_async_copy(
                pp_hbm.at[pl.ds(0, _N)], ins[s].at[pl.ds(8, _N)],
                sis[s]).start()

        @pl.when(i > 0)
        def _():
            pltpu.make_async_copy(
                pp_hbm.at[pl.ds(a8, _CHUNK)], ins[s], sis[s]).start()

    def in_full_wait(t, s):
        i, a8, _ = row_params(t)

        @pl.when(i == 0)
        def _():
            pltpu.make_async_copy(
                pp_hbm.at[pl.ds(0, _N)], ins[s].at[pl.ds(8, _N)],
                sis[s]).wait()

        @pl.when(i > 0)
        def _():
            pltpu.make_async_copy(
                pp_hbm.at[pl.ds(a8, _CHUNK)], ins[s], sis[s]).wait()

    def in_half(t, s):
        _, a8, _ = row_params(t)
        return pltpu.make_async_copy(
            pp_hbm.at[pl.ds(a8 + _H, _H + 8)],
            ins[s].at[pl.ds(_H, _H + 8)], sis[s])

    def out_full(t, s):
        i, _, _ = row_params(t)
        return pltpu.make_async_copy(obs[s], u_hbm.at[i], sos[s])

    def out_half(t, s):
        i, _, _ = row_params(t)
        return pltpu.make_async_copy(
            obs[s].at[pl.ds(_H, _H)], u_hbm.at[i, pl.ds(_H, _H)], sos[s])

    def in_start(t, s):
        @pl.when(t < _THALF)
        def _():
            in_full(t, s)

        @pl.when(t >= _THALF)
        def _():
            in_half(t, s).start()

    def in_wait(t, s):
        @pl.when(t < _THALF)
        def _():
            in_full_wait(t, s)

        @pl.when(t >= _THALF)
        def _():
            in_half(t, s).wait()

    def out_start(t, s):
        @pl.when(t < _THALF)
        def _():
            out_full(t, s).start()

        @pl.when(t >= _THALF)
        def _():
            out_half(t, s).start()

    def out_wait(t, s):
        @pl.when(t < _THALF)
        def _():
            out_full(t, s).wait()

        @pl.when(t >= _THALF)
        def _():
            out_half(t, s).wait()

    def shift(t, s):
        i, _, r = row_params(t)
        ib = ins[s]
        ob = obs[s]
        kb0 = (i + 1) // 256

        def blk(kb, c2):
            base = kb * 256
            vals = [ib[pl.ds(r + base + u * 16, 16)] for u in range(16)]
            for u, v in enumerate(vals):
                ob[pl.ds(base + u * 16, 16)] = v
            return c2

        lax.fori_loop(kb0, _N // 256, blk, 0)

    for t in range(_DEPTH - 1):
        in_start(t, t)

    def body(t4, carry):
        for s in range(_DEPTH):
            t = _DEPTH * t4 + s

            @pl.when(t + _DEPTH - 1 < _RPW)
            def _():
                in_start(t + _DEPTH - 1, (s + _DEPTH - 1) % _DEPTH)

            in_wait(t, s)

            @pl.when(t >= _DEPTH)
            def _():
                out_wait(t - _DEPTH, s)

            shift(t, s)
            out_start(t, s)
        return carry

    lax.fori_loop(0, _RPW // _DEPTH, body, 0)
    for t in range(_RPW - _DEPTH, _RPW):
        out_wait(t, t % _DEPTH)


@functools.partial(
    pl.kernel,
    out_type=jax.ShapeDtypeStruct((_N, _N), jnp.float32),
    mesh=plsc.VectorSubcoreMesh(core_axis_name="c", subcore_axis_name="s"),
    scratch_types=(
        [pltpu.VMEM((_CHUNK,), jnp.float32)] * _DEPTH
        + [pltpu.VMEM((_N,), jnp.float32)] * _DEPTH
        + [pltpu.SemaphoreType.DMA] * (2 * _DEPTH)
    ),
)
def _phase1(pp_hbm, u_hbm, *bufs):
    ins = bufs[0:_DEPTH]
    obs = bufs[_DEPTH:2 * _DEPTH]
    sis = bufs[2 * _DEPTH:3 * _DEPTH]
    sos = bufs[3 * _DEPTH:4 * _DEPTH]
    _phase1_body(pp_hbm, u_hbm, ins, obs, sis, sos)


def _phase2_body(bi_ref, bj_ref, u_ref, o_ref):
    g = pl.program_id(0)
    h = pl.program_id(1)
    bi = bi_ref[g]
    bj = bj_ref[g]
    ua = u_ref[...]
    ir = lax.broadcasted_iota(jnp.int32, (_B, _B), 0)
    ic = lax.broadcasted_iota(jnp.int32, (_B, _B), 1)
    zero = jnp.float32(0.0)

    @pl.when(h == 0)
    def _():
        gr = ir + bi * _B
        gc = ic + bj * _B
        o_ref[...] = jnp.where(gc > gr, ua, zero)

    @pl.when(h == 1)
    def _():
        gr = ir + bj * _B
        gc = ic + bi * _B
        uat = ua.T
        o_ref[...] = jnp.where(gc < gr, -uat, jnp.where(gc > gr, ua, zero))


def _phase2(u):
    pairs = [(a, b) for a in range(_NB) for b in range(a, _NB)]
    bi = jnp.asarray(np.array([p[0] for p in pairs], dtype=np.int32))
    bj = jnp.asarray(np.array([p[1] for p in pairs], dtype=np.int32))
    grid_spec = pltpu.PrefetchScalarGridSpec(
        num_scalar_prefetch=2,
        grid=(_NPAIR, 2),
        in_specs=[
            pl.BlockSpec((_B, _B), lambda g, h, bi, bj: (bi[g], bj[g])),
        ],
        out_specs=pl.BlockSpec(
            (_B, _B),
            lambda g, h, bi, bj: (
                jnp.where(h == 0, bi[g], bj[g]),
                jnp.where(h == 0, bj[g], bi[g]),
            ),
        ),
    )
    return pl.pallas_call(
        _phase2_body,
        grid_spec=grid_spec,
        out_shape=jax.ShapeDtypeStruct((_N, _N), jnp.float32),
    )(bi, bj, u)


def kernel(params, triu_indices):
    del triu_indices
    u = _phase1(params.astype(jnp.float32))
    return _phase2(u)


# speedup vs baseline: 298.9761x; 1.1603x over previous
import functools

import jax
import jax.numpy as jnp
import numpy as np
from jax import lax
from jax.experimental import pallas as pl
from jax.experimental.pallas import tpu as pltpu
from jax.experimental.pallas import tpu_sc as plsc

_N = 2048
_H = _N // 2
_M = _N * (_N - 1) // 2
_CHUNK = _N + 8
_NC = 2
_NS = 16
_NW = _NC * _NS
_RPW = _N // _NW
_DEPTH = 4
_B = 512
_NB = _N // _B
_NPAIR = _NB * (_NB + 1) // 2
_THALF = _H // _NW


def _phase1_body(pp_hbm, u_hbm, ins, obs, sis, sos):
    wid = lax.axis_index("s") * _NC + lax.axis_index("c")

    def row_params(t):
        i = t * _NW + wid
        off = i * (_N - 1) - (i * (i - 1)) // 2
        start = off - i - 1
        a8 = pl.multiple_of(
            jnp.clip((start // 8) * 8, 0, _M - _CHUNK).astype(jnp.int32), 8)
        r = jnp.where(i == 0, 7, start - a8)
        return i, a8, r

    def in_full(t, s):
        i, a8, _ = row_params(t)

        @pl.when(i == 0)
        def _():
            pltpu.make_async_copy(
                pp_hbm.at[pl.ds(0, _N)], ins[s].at[pl.ds(8, _N)],
                sis[s]).start()

        @pl.when(i > 0)
        def _():
            pltpu.make_async_copy(
                pp_hbm.at[pl.ds(a8, _CHUNK)], ins[s], sis[s]).start()

    def in_full_wait(t, s):
        i, a8, _ = row_params(t)

        @pl.when(i == 0)
        def _():
            pltpu.make_async_copy(
                pp_hbm.at[pl.ds(0, _N)], ins[s].at[pl.ds(8, _N)],
                sis[s]).wait()

        @pl.when(i > 0)
        def _():
            pltpu.make_async_copy(
                pp_hbm.at[pl.ds(a8, _CHUNK)], ins[s], sis[s]).wait()

    def in_half(t, s):
        _, a8, _ = row_params(t)
        return pltpu.make_async_copy(
            pp_hbm.at[pl.ds(a8 + _H, _H + 8)],
            ins[s].at[pl.ds(_H, _H + 8)], sis[s])

    def out_full(t, s):
        i, _, _ = row_params(t)
        return pltpu.make_async_copy(obs[s], u_hbm.at[i], sos[s])

    def out_half(t, s):
        i, _, _ = row_params(t)
        return pltpu.make_async_copy(
            obs[s].at[pl.ds(_H, _H)], u_hbm.at[i, pl.ds(_H, _H)], sos[s])

    def in_start(t, s):
        @pl.when(t < _THALF)
        def _():
            in_full(t, s)

        @pl.when(t >= _THALF)
        def _():
            in_half(t, s).start()

    def in_wait(t, s):
        @pl.when(t < _THALF)
        def _():
            in_full_wait(t, s)

        @pl.when(t >= _THALF)
        def _():
            in_half(t, s).wait()

    def out_start(t, s):
        @pl.when(t < _THALF)
        def _():
            out_full(t, s).start()

        @pl.when(t >= _THALF)
        def _():
            out_half(t, s).start()

    def out_wait(t, s):
        @pl.when(t < _THALF)
        def _():
            out_full(t, s).wait()

        @pl.when(t >= _THALF)
        def _():
            out_half(t, s).wait()

    def shift(t, s):
        i, _, r = row_params(t)
        ib = ins[s]
        ob = obs[s]
        kb0 = (i + 1) // 256

        def blk(kb, c2):
            base = kb * 256
            vals = [ib[pl.ds(r + base + u * 16, 16)] for u in range(16)]
            for u, v in enumerate(vals):
                ob[pl.ds(base + u * 16, 16)] = v
            return c2

        lax.fori_loop(kb0, _N // 256, blk, 0)

    for t in range(_DEPTH - 1):
        in_start(t, t)

    def body(t4, carry):
        for s in range(_DEPTH):
            t = _DEPTH * t4 + s

            @pl.when(t + _DEPTH - 1 < _RPW)
            def _():
                in_start(t + _DEPTH - 1, (s + _DEPTH - 1) % _DEPTH)

            in_wait(t, s)

            @pl.when(t >= _DEPTH)
            def _():
                out_wait(t - _DEPTH, s)

            shift(t, s)
            out_start(t, s)
        return carry

    lax.fori_loop(0, _RPW // _DEPTH, body, 0)
    for t in range(_RPW - _DEPTH, _RPW):
        out_wait(t, t % _DEPTH)


@functools.partial(
    pl.kernel,
    out_type=jax.ShapeDtypeStruct((_N, _N), jnp.float32),
    mesh=plsc.VectorSubcoreMesh(core_axis_name="c", subcore_axis_name="s"),
    scratch_types=(
        [pltpu.VMEM((_CHUNK,), jnp.float32)] * _DEPTH
        + [pltpu.VMEM((_N,), jnp.float32)] * _DEPTH
        + [pltpu.SemaphoreType.DMA] * (2 * _DEPTH)
    ),
)
def _phase1(pp_hbm, u_hbm, *bufs):
    ins = bufs[0:_DEPTH]
    obs = bufs[_DEPTH:2 * _DEPTH]
    sis = bufs[2 * _DEPTH:3 * _DEPTH]
    sos = bufs[3 * _DEPTH:4 * _DEPTH]
    _phase1_body(pp_hbm, u_hbm, ins, obs, sis, sos)


def _phase2_body(bi_ref, bj_ref, a_ref, o_ref):
    g = pl.program_id(0)
    bi = bi_ref[g]
    bj = bj_ref[g]
    ua = a_ref[...]
    ir = lax.broadcasted_iota(jnp.int32, (_B, _B), 0)
    ic = lax.broadcasted_iota(jnp.int32, (_B, _B), 1)
    zero = jnp.float32(0.0)
    gr = ir + bj * _B
    gc = ic + bi * _B
    uat = ua.T
    o_ref[...] = jnp.where(gc < gr, -uat, jnp.where(gc > gr, ua, zero))


def _phase2(a):
    pairs = [(x, y) for x in range(_NB) for y in range(x, _NB)]
    bi = jnp.asarray(np.array([p[0] for p in pairs], dtype=np.int32))
    bj = jnp.asarray(np.array([p[1] for p in pairs], dtype=np.int32))
    grid_spec = pltpu.PrefetchScalarGridSpec(
        num_scalar_prefetch=2,
        grid=(_NPAIR,),
        in_specs=[
            pl.BlockSpec((_B, _B), lambda g, bi, bj: (bi[g], bj[g])),
        ],
        out_specs=pl.BlockSpec((_B, _B), lambda g, bi, bj: (bj[g], bi[g])),
    )
    return pl.pallas_call(
        _phase2_body,
        grid_spec=grid_spec,
        out_shape=jax.ShapeDtypeStruct((_N, _N), jnp.float32),
        input_output_aliases={2: 0},
    )(bi, bj, a)


def kernel(params, triu_indices):
    del triu_indices
    a = _phase1(params.astype(jnp.float32))
    return _phase2(a)


# speedup vs baseline: 300.9870x; 1.0067x over previous
import functools

import jax
import jax.numpy as jnp
import numpy as np
from jax import lax
from jax.experimental import pallas as pl
from jax.experimental.pallas import tpu as pltpu
from jax.experimental.pallas import tpu_sc as plsc

_N = 2048
_M = _N * (_N - 1) // 2
_CHUNK = _N + 8
_Q = 512
_NC = 2
_NS = 16
_NW = _NC * _NS
_RPW = _N // _NW
_DEPTH = 4
_B = 512
_NB = _N // _B
_NPAIR = _NB * (_NB + 1) // 2


def _phase1_body(pp_hbm, u_hbm, ins, obs, sis, sos):
    wid = lax.axis_index("s") * _NC + lax.axis_index("c")

    def row_params(t):
        i = t * _NW + wid
        off = i * (_N - 1) - (i * (i - 1)) // 2
        start = off - i - 1
        a8 = pl.multiple_of(
            jnp.clip((start // 8) * 8, 0, _M - _CHUNK).astype(jnp.int32), 8)
        r = jnp.where(i == 0, 7, start - a8)
        return i, a8, r

    def in_copy(t, s, q):
        _, a8, _ = row_params(t)
        sz = _N - q * _Q + 8
        return pltpu.make_async_copy(
            pp_hbm.at[pl.ds(a8 + q * _Q, sz)],
            ins[s].at[pl.ds(q * _Q, sz)], sis[s])

    def in_row0(t, s):
        return pltpu.make_async_copy(
            pp_hbm.at[pl.ds(0, _N)], ins[s].at[pl.ds(8, _N)], sis[s])

    def out_copy(t, s, q):
        i, _, _ = row_params(t)
        sz = _N - q * _Q
        return pltpu.make_async_copy(
            obs[s].at[pl.ds(q * _Q, sz)],
            u_hbm.at[i, pl.ds(q * _Q, sz)], sos[s])

    def _dispatch_in(t, s, act):
        i, _, _ = row_params(t)
        iq = i // _Q
        for q in range(_N // _Q):
            cond = jnp.logical_and(iq == q, i > 0) if q == 0 else iq == q

            @pl.when(cond)
            def _():
                act(in_copy(t, s, q))

        @pl.when(i == 0)
        def _():
            act(in_row0(t, s))

    def _dispatch_out(t, s, act):
        i, _, _ = row_params(t)
        iq = i // _Q
        for q in range(_N // _Q):
            @pl.when(iq == q)
            def _():
                act(out_copy(t, s, q))

    def in_start(t, s):
        _dispatch_in(t, s, lambda c: c.start())

    def in_wait(t, s):
        _dispatch_in(t, s, lambda c: c.wait())

    def out_start(t, s):
        _dispatch_out(t, s, lambda c: c.start())

    def out_wait(t, s):
        _dispatch_out(t, s, lambda c: c.wait())

    def shift(t, s):
        i, _, r = row_params(t)
        ib = ins[s]
        ob = obs[s]
        kb0 = (i + 1) // 256

        def blk(kb, c2):
            base = kb * 256
            vals = [ib[pl.ds(r + base + u * 16, 16)] for u in range(16)]
            for u, v in enumerate(vals):
                ob[pl.ds(base + u * 16, 16)] = v
            return c2

        lax.fori_loop(kb0, _N // 256, blk, 0)

    for t in range(_DEPTH - 1):
        in_start(t, t)

    def body(t4, carry):
        for s in range(_DEPTH):
            t = _DEPTH * t4 + s

            @pl.when(t + _DEPTH - 1 < _RPW)
            def _():
                in_start(t + _DEPTH - 1, (s + _DEPTH - 1) % _DEPTH)

            in_wait(t, s)

            @pl.when(t >= _DEPTH)
            def _():
                out_wait(t - _DEPTH, s)

            shift(t, s)
            out_start(t, s)
        return carry

    lax.fori_loop(0, _RPW // _DEPTH, body, 0)
    for t in range(_RPW - _DEPTH, _RPW):
        out_wait(t, t % _DEPTH)


@functools.partial(
    pl.kernel,
    out_type=jax.ShapeDtypeStruct((_N, _N), jnp.float32),
    mesh=plsc.VectorSubcoreMesh(core_axis_name="c", subcore_axis_name="s"),
    scratch_types=(
        [pltpu.VMEM((_CHUNK,), jnp.float32)] * _DEPTH
        + [pltpu.VMEM((_N,), jnp.float32)] * _DEPTH
        + [pltpu.SemaphoreType.DMA] * (2 * _DEPTH)
    ),
)
def _phase1(pp_hbm, u_hbm, *bufs):
    ins = bufs[0:_DEPTH]
    obs = bufs[_DEPTH:2 * _DEPTH]
    sis = bufs[2 * _DEPTH:3 * _DEPTH]
    sos = bufs[3 * _DEPTH:4 * _DEPTH]
    _phase1_body(pp_hbm, u_hbm, ins, obs, sis, sos)


def _phase2_body(bi_ref, bj_ref, a_ref, o_ref):
    g = pl.program_id(0)
    bi = bi_ref[g]
    bj = bj_ref[g]
    ua = a_ref[...]
    ir = lax.broadcasted_iota(jnp.int32, (_B, _B), 0)
    ic = lax.broadcasted_iota(jnp.int32, (_B, _B), 1)
    zero = jnp.float32(0.0)
    gr = ir + bj * _B
    gc = ic + bi * _B
    uat = ua.T
    o_ref[...] = jnp.where(gc < gr, -uat, jnp.where(gc > gr, ua, zero))


def _phase2(a):
    pairs = [(x, y) for x in range(_NB) for y in range(x, _NB)]
    bi = jnp.asarray(np.array([p[0] for p in pairs], dtype=np.int32))
    bj = jnp.asarray(np.array([p[1] for p in pairs], dtype=np.int32))
    grid_spec = pltpu.PrefetchScalarGridSpec(
        num_scalar_prefetch=2,
        grid=(_NPAIR,),
        in_specs=[
            pl.BlockSpec((_B, _B), lambda g, bi, bj: (bi[g], bj[g])),
        ],
        out_specs=pl.BlockSpec((_B, _B), lambda g, bi, bj: (bj[g], bi[g])),
    )
    return pl.pallas_call(
        _phase2_body,
        grid_spec=grid_spec,
        out_shape=jax.ShapeDtypeStruct((_N, _N), jnp.float32),
        input_output_aliases={2: 0},
    )(bi, bj, a)


def kernel(params, triu_indices):
    del triu_indices
    a = _phase1(params.astype(jnp.float32))
    return _phase2(a)
